# baseline (speedup 1.0000x reference)
import functools

import jax
import jax.numpy as jnp
from jax import lax
from jax.experimental import pallas as pl
from jax.experimental.pallas import tpu as pltpu
from jax.experimental.pallas import tpu_sc as plsc

H = 128
H2 = 256
N = 10000
E = 320000

NC = 2
NS = 16
NW = NC * NS
CB = 100
CPW = 100
ACC_ROWS = 10240
ZROWS = 80
ZCHUNK = ACC_ROWS // NS // ZROWS
OUT_ROWS_PER_TILE = ACC_ROWS // NS

NBUF = 2
IB = 20

_PERM = []
for _g in range(H // 32):
    _PERM.extend(32 * _g + 2 * _i for _i in range(16))
    _PERM.extend(32 * _g + 2 * _i + 1 for _i in range(16))
_MASK_HI = jnp.int32(-65536)


def _sc_agg_body(x_hbm, src_hbm, dst_hbm, out_hbm, src_v, dst_v, acc,
                 *bufs_and_sems):
    frows = bufs_and_sems[:NBUF]
    ibufs = bufs_and_sems[NBUF:2 * NBUF]
    gsems = bufs_and_sems[2 * NBUF:3 * NBUF]
    ssems = bufs_and_sems[3 * NBUF:]
    c = lax.axis_index("c")
    s = lax.axis_index("s")
    wid = s * NC + c

    zv = jnp.zeros((16,), jnp.float32)

    def zrow(r, carry):
        for k in range(H // 16):
            frows[0][r, pl.ds(k * 16, 16)] = zv
        return carry

    lax.fori_loop(0, ZROWS, zrow, 0)
    for k in range(ZCHUNK):
        pltpu.sync_copy(frows[0].at[pl.ds(0, ZROWS)],
                        acc.at[pl.ds(s * ZCHUNK * ZROWS + k * ZROWS, ZROWS)])

    plsc.subcore_barrier()

    def convert(b):
        def crow(r, carry):
            for g in range(H // 32):
                v = ibufs[b][r, pl.ds(16 * g, 16)]
                lo = lax.bitcast_convert_type(lax.shift_left(v, 16), jnp.float32)
                hi = lax.bitcast_convert_type(lax.bitwise_and(v, _MASK_HI), jnp.float32)
                frows[b][r, pl.ds(32 * g, 16)] = lo
                frows[b][r, pl.ds(32 * g + 16, 16)] = hi
            return carry

        lax.fori_loop(0, CB, crow, 0)

    def outer(t, carry):
        pltpu.sync_copy(src_hbm.at[wid, pl.ds(t * IB, IB)], src_v)
        pltpu.sync_copy(dst_hbm.at[wid, pl.ds(t * IB, IB)], dst_v)
        for b in range(NBUF):
            pltpu.async_copy(x_hbm.at[src_v.at[b]], ibufs[b], gsems[b])

        def inner(u, carry2):
            j0 = u * NBUF
            for b in range(NBUF):
                j = j0 + b
                pltpu.make_async_copy(x_hbm.at[src_v.at[j]], ibufs[b],
                                      gsems[b]).wait()

                @pl.when(u > 0)
                def _():
                    pltpu.make_async_copy(frows[b], acc.at[dst_v.at[0]],
                                          ssems[b]).wait()

                convert(b)

                @pl.when(j + NBUF < IB)
                def _():
                    pltpu.async_copy(x_hbm.at[src_v.at[j + NBUF]], ibufs[b],
                                     gsems[b])

                pltpu.async_copy(frows[b], acc.at[dst_v.at[j]], ssems[b],
                                 add=True)
            return carry2

        lax.fori_loop(0, IB // NBUF, inner, 0)
        for b in range(NBUF):
            pltpu.make_async_copy(frows[b], acc.at[dst_v.at[0]],
                                  ssems[b]).wait()
        return carry

    lax.fori_loop(0, CPW // IB, outer, 0)

    plsc.subcore_barrier()
    pltpu.sync_copy(
        acc.at[pl.ds(s * OUT_ROWS_PER_TILE, OUT_ROWS_PER_TILE)],
        out_hbm.at[c, pl.ds(s * OUT_ROWS_PER_TILE, OUT_ROWS_PER_TILE)],
    )


def _make_sc_agg():
    mesh = plsc.VectorSubcoreMesh(core_axis_name="c", subcore_axis_name="s")
    return pl.kernel(
        _sc_agg_body,
        out_type=jax.ShapeDtypeStruct((NC, ACC_ROWS, H), jnp.float32),
        mesh=mesh,
        compiler_params=pltpu.CompilerParams(use_tc_tiling_on_sc=False),
        scratch_types=[
            pltpu.VMEM((IB, CB), jnp.int32),
            pltpu.VMEM((IB, CB), jnp.int32),
            pltpu.VMEM_SHARED((ACC_ROWS, H), jnp.float32),
        ] + [pltpu.VMEM((CB, H), jnp.float32) for _ in range(NBUF)]
          + [pltpu.VMEM((CB, H // 2), jnp.int32) for _ in range(NBUF)]
          + [pltpu.SemaphoreType.DMA for _ in range(2 * NBUF)],
    )


def _mlp_body(p_ref, x_ref, w1_ref, g1_ref, be_ref, w2_ref, b2_ref, o_ref):
    w1 = w1_ref[...]
    hn = p_ref[0, :N] + p_ref[1, :N]
    s_sum = jnp.sum(hn, axis=0, keepdims=True)
    gram = lax.dot_general(hn, hn, (((0,), (0,)), ((), ())),
                           preferred_element_type=jnp.float32)
    mean_hn = s_sum * (1.0 / N)
    mu1 = jnp.dot(mean_hn, w1, preferred_element_type=jnp.float32)
    gw = jnp.dot(gram, w1, preferred_element_type=jnp.float32)
    m2 = jnp.sum(w1 * gw, axis=0, keepdims=True) * (1.0 / N)
    var = m2 - mu1 * mu1
    inv = lax.rsqrt(var + 1e-5)
    a = inv * g1_ref[...]
    sh = be_ref[...] - mu1 * a

    h = jnp.dot(hn, w1, preferred_element_type=jnp.float32)
    h = jnp.maximum(h * a + sh, 0.0)
    o_ref[...] = (x_ref[...] + b2_ref[...]
                  + jnp.dot(h, w2_ref[...], preferred_element_type=jnp.float32))


def kernel(x, edge_index, W1, b1, gamma1, beta1, W2, b2):
    del b1
    ei = edge_index.astype(jnp.int32)
    src_p = ei[0].reshape(NW, CPW, CB)
    dst_p = ei[1].reshape(NW, CPW, CB)
    xb = lax.bitcast_convert_type(
        x.astype(jnp.bfloat16).reshape(N, H // 2, 2), jnp.int32)
    W1p = W1[jnp.array(_PERM, jnp.int32), :]

    partials = _make_sc_agg()(xb, src_p, dst_p)

    out = pl.pallas_call(
        _mlp_body,
        out_shape=jax.ShapeDtypeStruct((N, H), jnp.float32),
    )(partials, x, W1p, gamma1.reshape(1, H2), beta1.reshape(1, H2),
      W2, b2.reshape(1, H))

    return out


# speedup vs baseline: 1.0273x; 1.0273x over previous
import functools

import jax
import jax.numpy as jnp
from jax import lax
from jax.experimental import pallas as pl
from jax.experimental.pallas import tpu as pltpu
from jax.experimental.pallas import tpu_sc as plsc

H = 128
H2 = 256
N = 10000
E = 320000

NC = 2
NS = 16
NW = NC * NS
CB = 100
CPW = 100
ACC_ROWS = 10240
ZROWS = 80
ZCHUNK = ACC_ROWS // NS // ZROWS
OUT_ROWS_PER_TILE = ACC_ROWS // NS

NBUF = 2
IB = 50

_PERM = []
for _g in range(H // 32):
    _PERM.extend(32 * _g + 2 * _i for _i in range(16))
    _PERM.extend(32 * _g + 2 * _i + 1 for _i in range(16))
_MASK_HI = jnp.int32(-65536)


def _sc_agg_body(x_hbm, src_hbm, dst_hbm, out_hbm, src_v, dst_v, acc,
                 *bufs_and_sems):
    frows = bufs_and_sems[:NBUF]
    ibufs = bufs_and_sems[NBUF:2 * NBUF]
    gsems = bufs_and_sems[2 * NBUF:3 * NBUF]
    ssems = bufs_and_sems[3 * NBUF:]
    c = lax.axis_index("c")
    s = lax.axis_index("s")
    wid = s * NC + c

    zv = jnp.zeros((16,), jnp.float32)

    def zrow(r, carry):
        for k in range(H // 16):
            frows[0][r, pl.ds(k * 16, 16)] = zv
        return carry

    lax.fori_loop(0, ZROWS, zrow, 0)
    for k in range(ZCHUNK):
        pltpu.sync_copy(frows[0].at[pl.ds(0, ZROWS)],
                        acc.at[pl.ds(s * ZCHUNK * ZROWS + k * ZROWS, ZROWS)])

    plsc.subcore_barrier()

    def convert(b):
        def crow(r, carry):
            for g in range(H // 32):
                v = ibufs[b][r, pl.ds(16 * g, 16)]
                lo = lax.bitcast_convert_type(lax.shift_left(v, 16), jnp.float32)
                hi = lax.bitcast_convert_type(lax.bitwise_and(v, _MASK_HI), jnp.float32)
                frows[b][r, pl.ds(32 * g, 16)] = lo
                frows[b][r, pl.ds(32 * g + 16, 16)] = hi
            return carry

        lax.fori_loop(0, CB, crow, 0)

    def outer(t, carry):
        pltpu.sync_copy(src_hbm.at[wid, pl.ds(t * IB, IB)], src_v)
        pltpu.sync_copy(dst_hbm.at[wid, pl.ds(t * IB, IB)], dst_v)
        for b in range(NBUF):
            pltpu.async_copy(x_hbm.at[src_v.at[b]], ibufs[b], gsems[b])

        def inner(u, carry2):
            j0 = u * NBUF
            for b in range(NBUF):
                j = j0 + b
                pltpu.make_async_copy(x_hbm.at[src_v.at[j]], ibufs[b],
                                      gsems[b]).wait()

                @pl.when(u > 0)
                def _():
                    pltpu.make_async_copy(frows[b], acc.at[dst_v.at[0]],
                                          ssems[b]).wait()

                convert(b)

                @pl.when(j + NBUF < IB)
                def _():
                    pltpu.async_copy(x_hbm.at[src_v.at[j + NBUF]], ibufs[b],
                                     gsems[b])

                pltpu.async_copy(frows[b], acc.at[dst_v.at[j]], ssems[b],
                                 add=True)
            return carry2

        lax.fori_loop(0, IB // NBUF, inner, 0)
        for b in range(NBUF):
            pltpu.make_async_copy(frows[b], acc.at[dst_v.at[0]],
                                  ssems[b]).wait()
        return carry

    lax.fori_loop(0, CPW // IB, outer, 0)

    plsc.subcore_barrier()
    pltpu.sync_copy(
        acc.at[pl.ds(s * OUT_ROWS_PER_TILE, OUT_ROWS_PER_TILE)],
        out_hbm.at[c, pl.ds(s * OUT_ROWS_PER_TILE, OUT_ROWS_PER_TILE)],
    )


def _make_sc_agg():
    mesh = plsc.VectorSubcoreMesh(core_axis_name="c", subcore_axis_name="s")
    return pl.kernel(
        _sc_agg_body,
        out_type=jax.ShapeDtypeStruct((NC, ACC_ROWS, H), jnp.float32),
        mesh=mesh,
        compiler_params=pltpu.CompilerParams(use_tc_tiling_on_sc=False),
        scratch_types=[
            pltpu.VMEM((IB, CB), jnp.int32),
            pltpu.VMEM((IB, CB), jnp.int32),
            pltpu.VMEM_SHARED((ACC_ROWS, H), jnp.float32),
        ] + [pltpu.VMEM((CB, H), jnp.float32) for _ in range(NBUF)]
          + [pltpu.VMEM((CB, H // 2), jnp.int32) for _ in range(NBUF)]
          + [pltpu.SemaphoreType.DMA for _ in range(2 * NBUF)],
    )


def _mlp_body(p_ref, x_ref, w1_ref, g1_ref, be_ref, w2_ref, b2_ref, o_ref):
    w1 = w1_ref[...]
    hn = p_ref[0, :N] + p_ref[1, :N]
    s_sum = jnp.sum(hn, axis=0, keepdims=True)
    gram = lax.dot_general(hn, hn, (((0,), (0,)), ((), ())),
                           preferred_element_type=jnp.float32)
    mean_hn = s_sum * (1.0 / N)
    mu1 = jnp.dot(mean_hn, w1, preferred_element_type=jnp.float32)
    gw = jnp.dot(gram, w1, preferred_element_type=jnp.float32)
    m2 = jnp.sum(w1 * gw, axis=0, keepdims=True) * (1.0 / N)
    var = m2 - mu1 * mu1
    inv = lax.rsqrt(var + 1e-5)
    a = inv * g1_ref[...]
    sh = be_ref[...] - mu1 * a

    h = jnp.dot(hn, w1, preferred_element_type=jnp.float32)
    h = jnp.maximum(h * a + sh, 0.0)
    o_ref[...] = (x_ref[...] + b2_ref[...]
                  + jnp.dot(h, w2_ref[...], preferred_element_type=jnp.float32))


def kernel(x, edge_index, W1, b1, gamma1, beta1, W2, b2):
    del b1
    ei = edge_index.astype(jnp.int32)
    src_p = ei[0].reshape(NW, CPW, CB)
    dst_p = ei[1].reshape(NW, CPW, CB)
    xb = lax.bitcast_convert_type(
        x.astype(jnp.bfloat16).reshape(N, H // 2, 2), jnp.int32)
    W1p = W1[jnp.array(_PERM, jnp.int32), :]

    partials = _make_sc_agg()(xb, src_p, dst_p)

    out = pl.pallas_call(
        _mlp_body,
        out_shape=jax.ShapeDtypeStruct((N, H), jnp.float32),
    )(partials, x, W1p, gamma1.reshape(1, H2), beta1.reshape(1, H2),
      W2, b2.reshape(1, H))

    return out
